# baseline (speedup 1.0000x reference)
import functools

import jax
import jax.numpy as jnp
from jax import lax
from jax.experimental import pallas as pl
from jax.experimental.pallas import tpu as pltpu
from jax.experimental.pallas import tpu_sc as plsc

BN_EPS = 1e-5
_NC = 2
_NS = 16
_NW = _NC * _NS
_CH = 128


def _pack_edges(src, dst, n_sc, dst_fill):
    e = src.shape[0]
    per_w = e // _NW
    assert per_w * _NW == e
    pad = n_sc * 2 * _CH - per_w

    def shape(a, fill):
        body = a.reshape(_NW, per_w)
        padb = jnp.full((_NW, pad), fill, dtype=a.dtype)
        return jnp.concatenate([body, padb],
                               axis=1).reshape(_NW, n_sc, 1, 2 * _CH)

    return jnp.concatenate([shape(src, 0), shape(dst, dst_fill)], axis=2)


def _sc_scatter_add(x, idx4, n_sc):
    n, d = x.shape
    assert idx4.shape == (_NW, n_sc, 2, 2 * _CH)
    rpt = (n // _NS) & ~7
    last = n - rpt * (_NS - 1)

    mesh = plsc.VectorSubcoreMesh(
        core_axis_name="c", subcore_axis_name="s", num_cores=_NC,
        num_subcores=_NS)

    scratch = [
        pltpu.VMEM((2 * _CH,), jnp.int32),
        pltpu.VMEM((2 * _CH,), jnp.int32),
        pltpu.VMEM((2 * _CH, d), jnp.float32),
        pltpu.VMEM_SHARED((n + 8, d), jnp.float32),
        pltpu.SemaphoreType.DMA,
    ]

    @functools.partial(
        pl.kernel, mesh=mesh,
        out_type=jax.ShapeDtypeStruct((_NC, n, d), jnp.float32),
        scratch_types=scratch,
    )
    def sc_body(x_hbm, idx_hbm, out_hbm, si, di, rows, agg, sem):
        cid = lax.axis_index("c")
        sid = lax.axis_index("s")
        wid = sid * _NC + cid

        @pl.when(sid < _NS - 1)
        def _():
            r0 = sid * rpt
            pltpu.sync_copy(x_hbm.at[pl.ds(r0, rpt)], agg.at[pl.ds(r0, rpt)])

        @pl.when(sid == _NS - 1)
        def _():
            r0 = (_NS - 1) * rpt
            pltpu.sync_copy(x_hbm.at[pl.ds(r0, last)], agg.at[pl.ds(r0, last)])

        plsc.subcore_barrier()

        def body(j, carry):
            pltpu.sync_copy(idx_hbm.at[wid, j, 0], si)
            pltpu.sync_copy(idx_hbm.at[wid, j, 1], di)
            pltpu.async_copy(x_hbm.at[si], rows, sem).wait()
            pltpu.sync_copy(rows, agg.at[di], add=True)
            return carry

        lax.fori_loop(0, n_sc, body, 0, unroll=False)

        plsc.subcore_barrier()

        @pl.when(sid < _NS - 1)
        def _():
            r0 = sid * rpt
            pltpu.sync_copy(agg.at[pl.ds(r0, rpt)],
                            out_hbm.at[cid, pl.ds(r0, rpt)])

        @pl.when(sid == _NS - 1)
        def _():
            r0 = (_NS - 1) * rpt
            pltpu.sync_copy(agg.at[pl.ds(r0, last)],
                            out_hbm.at[cid, pl.ds(r0, last)])

    return sc_body(x, idx4)


def _mlp_bn_relu(h0, Wa, ba, Wb, bb, g, be):
    hp = jax.lax.Precision.HIGHEST
    h = jnp.dot(h0, Wa, precision=hp) + ba
    h = jnp.maximum(h, 0.0)
    h = jnp.dot(h, Wb, precision=hp) + bb
    mean = jnp.mean(h, axis=0, keepdims=True)
    var = jnp.mean((h - mean) ** 2, axis=0, keepdims=True)
    h = g * (h - mean) / jnp.sqrt(var + BN_EPS) + be
    return jnp.maximum(h, 0.0)


def _tc_layer(x, agg, Wa, ba, Wb, bb, g, be):
    n, d = x.shape
    h = Wa.shape[1]

    def body(x_ref, agg_ref, wa, ba_r, wb, bb_r, g_r, be_r, o_ref):
        h0 = agg_ref[0] + agg_ref[1] - x_ref[...]
        o_ref[...] = _mlp_bn_relu(h0, wa[...], ba_r[...], wb[...], bb_r[...],
                                  g_r[...], be_r[...])

    return pl.pallas_call(
        body,
        out_shape=jax.ShapeDtypeStruct((n, h), jnp.float32),
    )(x, agg, Wa, ba, Wb, bb, g, be)


def _tc_final(x, agg, batch, Wa, ba, Wb, bb, g, be, Wl, bl, num_graphs):
    n, d = x.shape
    out_dim = Wl.shape[1]

    def body(x_ref, agg_ref, batch_ref, wa, ba_r, wb, bb_r, g_r, be_r,
             wl, bl_r, o_ref):
        h0 = agg_ref[0] + agg_ref[1] - x_ref[...]
        h2 = _mlp_bn_relu(h0, wa[...], ba_r[...], wb[...], bb_r[...],
                          g_r[...], be_r[...])
        seg = batch_ref[...]
        gids = lax.broadcasted_iota(jnp.int32, (num_graphs, n), 0)
        ind = (seg[None, :] == gids).astype(jnp.float32)
        hp = jax.lax.Precision.HIGHEST
        pooled = jnp.dot(ind, h2, precision=hp)
        o_ref[...] = jnp.dot(pooled, wl[...], precision=hp) + bl_r[...]

    return pl.pallas_call(
        body,
        out_shape=jax.ShapeDtypeStruct((num_graphs, out_dim), jnp.float32),
    )(x, agg, batch, Wa, ba, Wb, bb, g, be, Wl, bl)


def kernel(x, edge_index, batch, W1a, b1a, W1b, b1b, g1, be1, W2a, b2a, W2b,
           b2b, g2, be2, Wl, bl):
    n = x.shape[0]
    e = edge_index.shape[1]
    num_graphs = 64

    n_sc = -(-e // (_NW * 2 * _CH))
    idx4 = _pack_edges(edge_index[0], edge_index[1], n_sc, n)

    agg1 = _sc_scatter_add(x, idx4, n_sc)
    h1 = _tc_layer(x, agg1, W1a, b1a, W1b, b1b, g1, be1)
    agg2 = _sc_scatter_add(h1, idx4, n_sc)
    out = _tc_final(h1, agg2, batch, W2a, b2a, W2b, b2b, g2, be2, Wl, bl,
                    num_graphs)
    return out


# speedup vs baseline: 2.0657x; 2.0657x over previous
import functools

import jax
import jax.numpy as jnp
from jax import lax
from jax.experimental import pallas as pl
from jax.experimental.pallas import tpu as pltpu
from jax.experimental.pallas import tpu_sc as plsc

BN_EPS = 1e-5
_NC = 2
_NS = 16
_CH = 128


def _sc_scatter_add(x, src, dst):
    n, d = x.shape
    e = src.shape[0]
    nw = _NC * _NS
    epw = e // nw
    assert epw * nw == e and epw % 8 == 0
    n_full = epw // _CH
    tail = epw - n_full * _CH
    assert tail % 8 == 0
    rpt = (n // _NS) & ~7
    last = n - rpt * (_NS - 1)

    mesh = plsc.VectorSubcoreMesh(
        core_axis_name="c", subcore_axis_name="s", num_cores=_NC,
        num_subcores=_NS)

    scratch = [
        pltpu.VMEM((_CH,), jnp.int32),
        pltpu.VMEM((_CH,), jnp.int32),
        pltpu.VMEM((_CH, d), jnp.float32),
        pltpu.VMEM_SHARED((n, d), jnp.float32),
        pltpu.SemaphoreType.DMA,
    ]
    if tail:
        scratch += [
            pltpu.VMEM((tail,), jnp.int32),
            pltpu.VMEM((tail,), jnp.int32),
            pltpu.VMEM((tail, d), jnp.float32),
        ]

    @functools.partial(
        pl.kernel, mesh=mesh,
        out_type=jax.ShapeDtypeStruct((_NC, n, d), jnp.float32),
        scratch_types=scratch,
    )
    def sc_body(x_hbm, src_hbm, dst_hbm, out_hbm, si, di, rows, agg, sem,
                *tail_bufs):
        cid = lax.axis_index("c")
        sid = lax.axis_index("s")
        wid = sid * _NC + cid

        @pl.when(sid < _NS - 1)
        def _():
            r0 = sid * rpt
            pltpu.sync_copy(x_hbm.at[pl.ds(r0, rpt)], agg.at[pl.ds(r0, rpt)])

        @pl.when(sid == _NS - 1)
        def _():
            r0 = (_NS - 1) * rpt
            pltpu.sync_copy(x_hbm.at[pl.ds(r0, last)], agg.at[pl.ds(r0, last)])

        plsc.subcore_barrier()

        base0 = wid * epw

        def body(i, carry):
            b = base0 + i * _CH
            pltpu.sync_copy(src_hbm.at[pl.ds(b, _CH)], si)
            dsc = pltpu.async_copy(x_hbm.at[si], rows, sem)
            pltpu.sync_copy(dst_hbm.at[pl.ds(b, _CH)], di)
            dsc.wait()
            pltpu.sync_copy(rows, agg.at[di], add=True)
            return carry

        lax.fori_loop(0, n_full, body, 0)
        if tail:
            sit, dit, rowst = tail_bufs
            b = base0 + n_full * _CH
            pltpu.sync_copy(src_hbm.at[pl.ds(b, tail)], sit)
            dsc = pltpu.async_copy(x_hbm.at[sit], rowst, sem)
            pltpu.sync_copy(dst_hbm.at[pl.ds(b, tail)], dit)
            dsc.wait()
            pltpu.sync_copy(rowst, agg.at[dit], add=True)
        plsc.subcore_barrier()

        @pl.when(sid < _NS - 1)
        def _():
            r0 = sid * rpt
            pltpu.sync_copy(agg.at[pl.ds(r0, rpt)],
                            out_hbm.at[cid, pl.ds(r0, rpt)])

        @pl.when(sid == _NS - 1)
        def _():
            r0 = (_NS - 1) * rpt
            pltpu.sync_copy(agg.at[pl.ds(r0, last)],
                            out_hbm.at[cid, pl.ds(r0, last)])

    return sc_body(x, src, dst)


def _mlp_bn_relu(h0, Wa, ba, Wb, bb, g, be):
    hp = jax.lax.Precision.HIGHEST
    h = jnp.dot(h0, Wa, precision=hp) + ba
    h = jnp.maximum(h, 0.0)
    h = jnp.dot(h, Wb, precision=hp) + bb
    mean = jnp.mean(h, axis=0, keepdims=True)
    var = jnp.mean((h - mean) ** 2, axis=0, keepdims=True)
    h = g * (h - mean) / jnp.sqrt(var + BN_EPS) + be
    return jnp.maximum(h, 0.0)


def _tc_layer(x, agg, Wa, ba, Wb, bb, g, be):
    n, d = x.shape
    h = Wa.shape[1]

    def body(x_ref, agg_ref, wa, ba_r, wb, bb_r, g_r, be_r, o_ref):
        h0 = agg_ref[0] + agg_ref[1] - x_ref[...]
        o_ref[...] = _mlp_bn_relu(h0, wa[...], ba_r[...], wb[...], bb_r[...],
                                  g_r[...], be_r[...])

    return pl.pallas_call(
        body,
        out_shape=jax.ShapeDtypeStruct((n, h), jnp.float32),
    )(x, agg, Wa, ba, Wb, bb, g, be)


def _tc_final(x, agg, batch, Wa, ba, Wb, bb, g, be, Wl, bl, num_graphs):
    n, d = x.shape
    out_dim = Wl.shape[1]

    def body(x_ref, agg_ref, batch_ref, wa, ba_r, wb, bb_r, g_r, be_r,
             wl, bl_r, o_ref):
        h0 = agg_ref[0] + agg_ref[1] - x_ref[...]
        h2 = _mlp_bn_relu(h0, wa[...], ba_r[...], wb[...], bb_r[...],
                          g_r[...], be_r[...])
        seg = batch_ref[...]
        gids = lax.broadcasted_iota(jnp.int32, (num_graphs, n), 0)
        ind = (seg[None, :] == gids).astype(jnp.float32)
        hp = jax.lax.Precision.HIGHEST
        pooled = jnp.dot(ind, h2, precision=hp)
        o_ref[...] = jnp.dot(pooled, wl[...], precision=hp) + bl_r[...]

    return pl.pallas_call(
        body,
        out_shape=jax.ShapeDtypeStruct((num_graphs, out_dim), jnp.float32),
    )(x, agg, batch, Wa, ba, Wb, bb, g, be, Wl, bl)


def kernel(x, edge_index, batch, W1a, b1a, W1b, b1b, g1, be1, W2a, b2a, W2b,
           b2b, g2, be2, Wl, bl):
    src = edge_index[0]
    dst = edge_index[1]
    num_graphs = 64

    agg1 = _sc_scatter_add(x, src, dst)
    h1 = _tc_layer(x, agg1, W1a, b1a, W1b, b1b, g1, be1)
    agg2 = _sc_scatter_add(h1, src, dst)
    out = _tc_final(h1, agg2, batch, W2a, b2a, W2b, b2b, g2, be2, Wl, bl,
                    num_graphs)
    return out
